# speedup vs baseline: 2.1828x; 2.1828x over previous
import jax
import jax.numpy as jnp
from jax import lax
from jax.experimental import pallas as pl
from jax.experimental.pallas import tpu as pltpu
from jax.experimental.pallas import tpu_sc as plsc

_NC = 2
_NS = 16
_NW = _NC * _NS

_N = 4 * 8192
_D = 1024
_PER_W = _N // _NW
_C = 64
_NCHUNK = _PER_W // _C


def _gather_body(idx_hbm, table_hbm, out_hbm, idx_v, rows_v, sem):
    wid = lax.axis_index("s") * _NC + lax.axis_index("c")
    base = wid * _PER_W
    pltpu.sync_copy(idx_hbm.at[wid], idx_v)

    def chunk(c, carry):
        pltpu.async_copy(table_hbm.at[idx_v.at[c]], rows_v, sem).wait()
        pltpu.sync_copy(rows_v, out_hbm.at[pl.ds(base + c * _C, _C)])
        return carry

    lax.fori_loop(0, _NCHUNK, chunk, 0)


@jax.jit
def _sc_gather(idx, table):
    mesh = plsc.VectorSubcoreMesh(core_axis_name="c", subcore_axis_name="s")
    return pl.kernel(
        _gather_body,
        out_type=jax.ShapeDtypeStruct((_N, _D), jnp.float32),
        mesh=mesh,
        scratch_types=[
            pltpu.VMEM((_NCHUNK, _C), jnp.int32),
            pltpu.VMEM((_C, _D), jnp.float32),
            pltpu.SemaphoreType.DMA,
        ],
    )(idx, table)


def kernel(position_ids, table):
    idx = position_ids.astype(jnp.int32).reshape(_NW, _NCHUNK, _C)
    out = _sc_gather(idx, table)
    return out.reshape(position_ids.shape + (table.shape[1],))


# speedup vs baseline: 2.3669x; 1.0843x over previous
import jax
import jax.numpy as jnp
from jax import lax
from jax.experimental import pallas as pl
from jax.experimental.pallas import tpu as pltpu
from jax.experimental.pallas import tpu_sc as plsc

_NC = 2
_NS = 16
_NW = _NC * _NS

_N = 4 * 8192
_D = 1024
_PER_W = _N // _NW
_C = 32
_NCHUNK = _PER_W // _C
_H = _NCHUNK // 2


def _gather_body(idx_hbm, table_hbm, out_hbm, idx_v, rows0, rows1, sem0, sem1):
    wid = lax.axis_index("s") * _NC + lax.axis_index("c")
    base = wid * _PER_W
    pltpu.sync_copy(idx_hbm.at[wid], idx_v)
    pltpu.async_copy(table_hbm.at[idx_v.at[0]], rows0, sem0)
    pltpu.async_copy(table_hbm.at[idx_v.at[1]], rows1, sem1)

    def pair(i, carry):
        c0 = i * 2
        pltpu.make_async_copy(table_hbm.at[pl.ds(0, _C)], rows0, sem0).wait()
        pltpu.sync_copy(rows0, out_hbm.at[pl.ds(base + c0 * _C, _C)])

        @pl.when(i < _H - 1)
        def _():
            pltpu.async_copy(table_hbm.at[idx_v.at[c0 + 2]], rows0, sem0)

        pltpu.make_async_copy(table_hbm.at[pl.ds(0, _C)], rows1, sem1).wait()
        pltpu.sync_copy(rows1, out_hbm.at[pl.ds(base + (c0 + 1) * _C, _C)])

        @pl.when(i < _H - 1)
        def _():
            pltpu.async_copy(table_hbm.at[idx_v.at[c0 + 3]], rows1, sem1)

        return carry

    lax.fori_loop(0, _H, pair, 0)


@jax.jit
def _sc_gather(idx, table):
    mesh = plsc.VectorSubcoreMesh(core_axis_name="c", subcore_axis_name="s")
    return pl.kernel(
        _gather_body,
        out_type=jax.ShapeDtypeStruct((_N, _D), jnp.float32),
        mesh=mesh,
        scratch_types=[
            pltpu.VMEM((_NCHUNK, _C), jnp.int32),
            pltpu.VMEM((_C, _D), jnp.float32),
            pltpu.VMEM((_C, _D), jnp.float32),
            pltpu.SemaphoreType.DMA,
            pltpu.SemaphoreType.DMA,
        ],
    )(idx, table)


def kernel(position_ids, table):
    idx = position_ids.astype(jnp.int32).reshape(_NW, _NCHUNK, _C)
    out = _sc_gather(idx, table)
    return out.reshape(position_ids.shape + (table.shape[1],))
